# baseline (speedup 1.0000x reference)
import functools

import jax
import jax.numpy as jnp
import numpy as np
from jax.experimental import pallas as pl

NBINS = 50
F_MIN = 0.0
SCALE = 0.02
PAD = -1.0
B = 64
T = 8192


def _gauss_taps():
    x = np.linspace(-2.0, 2.0, 5)
    w = np.exp(-0.5 * (x / 0.5) ** 2)
    w = w / w.sum()
    return [float(v) for v in w]


_TAPS = _gauss_taps()


def _loss_kernel(x_ref, g_ref, out_ref):
    g = g_ref[...]
    x = x_ref[...]
    q = jnp.clip(jnp.floor((g - F_MIN) / SCALE).astype(jnp.int32), 0, NBINS - 1)
    n = jax.lax.broadcasted_iota(jnp.int32, x.shape, 1)
    z = jnp.zeros_like(x)
    for i in range(5):
        idx = n - 2 + i
        ridx = jnp.where(idx < 0, -idx, jnp.where(idx > NBINS - 1, 2 * (NBINS - 1) - idx, idx))
        z = z + _TAPS[i] * (ridx == q).astype(jnp.float32)
    term = jnp.maximum(x, 0.0) - x * z + jnp.log1p(jnp.exp(-jnp.abs(x)))
    valid = g != PAD
    partial = jnp.sum(jnp.where(valid, term, 0.0)).reshape(1, 1)

    @pl.when(pl.program_id(0) == 0)
    def _():
        out_ref[...] = jnp.zeros_like(out_ref)

    out_ref[...] += partial


@jax.jit
def kernel(preds, gt):
    rows = B * T
    tt = 8192
    xr = preds.reshape(rows, NBINS)
    gr = gt.reshape(rows, 1)
    out = pl.pallas_call(
        _loss_kernel,
        grid=(rows // tt,),
        in_specs=[
            pl.BlockSpec((tt, NBINS), lambda i: (i, 0)),
            pl.BlockSpec((tt, 1), lambda i: (i, 0)),
        ],
        out_specs=pl.BlockSpec((1, 1), lambda i: (0, 0)),
        out_shape=jax.ShapeDtypeStruct((1, 1), jnp.float32),
    )(xr, gr)
    return out[0, 0]


# speedup vs baseline: 1.2296x; 1.2296x over previous
import jax
import jax.numpy as jnp
import numpy as np
from jax.experimental import pallas as pl

NBINS = 50
F_MIN = 0.0
SCALE = 0.02
PAD = -1.0
B = 64
T = 8192

BB = 8
TT = 1024


def _gauss_taps():
    x = np.linspace(-2.0, 2.0, 5)
    w = np.exp(-0.5 * (x / 0.5) ** 2)
    w = w / w.sum()
    return [float(v) for v in w]


_TAPS = _gauss_taps()


def _loss_kernel(x_ref, g_ref, out_ref):
    g = g_ref[...]
    x = x_ref[...]
    q = jnp.clip(jnp.floor((g - F_MIN) / SCALE).astype(jnp.int32), 0, NBINS - 1)
    q3 = q[..., None]
    n = jax.lax.broadcasted_iota(jnp.int32, x.shape, 2)
    z = jnp.zeros_like(x)
    for i in range(5):
        idx = n - 2 + i
        ridx = jnp.where(idx < 0, -idx, jnp.where(idx > NBINS - 1, 2 * (NBINS - 1) - idx, idx))
        z = z + _TAPS[i] * (ridx == q3).astype(jnp.float32)
    term = jnp.maximum(x, 0.0) - x * z + jnp.log1p(jnp.exp(-jnp.abs(x)))
    validf = (g != PAD).astype(jnp.float32)
    partial = jnp.sum(validf[..., None] * term).reshape(1, 1)

    @pl.when((pl.program_id(0) == 0) & (pl.program_id(1) == 0))
    def _():
        out_ref[...] = jnp.zeros_like(out_ref)

    out_ref[...] += partial


@jax.jit
def kernel(preds, gt):
    out = pl.pallas_call(
        _loss_kernel,
        grid=(B // BB, T // TT),
        in_specs=[
            pl.BlockSpec((BB, TT, NBINS), lambda i, j: (i, j, 0)),
            pl.BlockSpec((BB, TT), lambda i, j: (i, j)),
        ],
        out_specs=pl.BlockSpec((1, 1), lambda i, j: (0, 0)),
        out_shape=jax.ShapeDtypeStruct((1, 1), jnp.float32),
    )(preds, gt)
    return out[0, 0]


# speedup vs baseline: 1.3778x; 1.1206x over previous
import jax
import jax.numpy as jnp
import numpy as np
from jax.experimental import pallas as pl

NBINS = 50
F_MIN = 0.0
SCALE = 0.02
PAD = -1.0
B = 64
T = 8192

BB = 8
TT = 1024


def _blur_table():
    x = np.linspace(-2.0, 2.0, 5)
    w = np.exp(-0.5 * (x / 0.5) ** 2)
    w = (w / w.sum()).astype(np.float32)
    tab = np.zeros((NBINS, NBINS), dtype=np.float32)
    for q in range(NBINS):
        for n in range(NBINS):
            acc = np.float32(0.0)
            for i in range(5):
                m = n - 2 + i
                r = -m if m < 0 else (2 * (NBINS - 1) - m if m > NBINS - 1 else m)
                if r == q:
                    acc += w[i]
            tab[q, n] = acc
    return tab


_W = _blur_table()


def _loss_kernel(x_ref, g_ref, w_ref, out_ref):
    g = g_ref[...]
    x = x_ref[...]
    q = jnp.clip(jnp.floor((g - F_MIN) / SCALE).astype(jnp.int32), 0, NBINS - 1)
    q3 = q[..., None]
    validf = (g != PAD).astype(jnp.float32)
    n = jax.lax.broadcasted_iota(jnp.int32, x.shape, 2)
    onehot = (n == q3).astype(jnp.float32) * validf[..., None]
    sp = jnp.maximum(x, 0.0) + jnp.log1p(jnp.exp(-jnp.abs(x)))
    sp_sum = jnp.sum(validf[..., None] * sp)

    x2 = x.reshape(BB * TT, NBINS)
    o2 = onehot.reshape(BB * TT, NBINS)
    gmat = jax.lax.dot_general(
        o2, x2, (((0,), (0,)), ((), ())), preferred_element_type=jnp.float32
    )
    g_term = jnp.sum(gmat * w_ref[...])
    partial = (sp_sum - g_term).reshape(1, 1)

    @pl.when((pl.program_id(0) == 0) & (pl.program_id(1) == 0))
    def _():
        out_ref[...] = jnp.zeros_like(out_ref)

    out_ref[...] += partial


@jax.jit
def kernel(preds, gt):
    out = pl.pallas_call(
        _loss_kernel,
        grid=(B // BB, T // TT),
        in_specs=[
            pl.BlockSpec((BB, TT, NBINS), lambda i, j: (i, j, 0)),
            pl.BlockSpec((BB, TT), lambda i, j: (i, j)),
            pl.BlockSpec((NBINS, NBINS), lambda i, j: (0, 0)),
        ],
        out_specs=pl.BlockSpec((1, 1), lambda i, j: (0, 0)),
        out_shape=jax.ShapeDtypeStruct((1, 1), jnp.float32),
    )(preds, gt, jnp.asarray(_W))
    return out[0, 0]


# speedup vs baseline: 5.1987x; 3.7731x over previous
import jax
import jax.numpy as jnp
import numpy as np
from jax.experimental import pallas as pl

NBINS = 50
F_MIN = 0.0
INV_SCALE = 50.0
PAD = -1.0
B = 64
T = 8192

TT = 512


def _blur_table():
    x = np.linspace(-2.0, 2.0, 5)
    w = np.exp(-0.5 * (x / 0.5) ** 2)
    w = (w / w.sum()).astype(np.float32)
    tab = np.zeros((NBINS, NBINS), dtype=np.float32)
    for q in range(NBINS):
        for n in range(NBINS):
            acc = np.float32(0.0)
            for i in range(5):
                m = n - 2 + i
                r = -m if m < 0 else (2 * (NBINS - 1) - m if m > NBINS - 1 else m)
                if r == q:
                    acc += w[i]
            tab[q, n] = acc
    return tab


_W = _blur_table()
_PLANE_TAPS = [
    [(int(a), float(_W[a, n])) for a in range(NBINS) if _W[a, n] != 0.0]
    for n in range(NBINS)
]


def _loss_kernel(x_ref, g_ref, out_ref):
    g = g_ref[...]
    q = jnp.clip(jnp.floor((g - F_MIN) * INV_SCALE).astype(jnp.int32), 0, NBINS - 1)
    validf = (g != PAD).astype(jnp.float32)
    inner = jnp.zeros_like(g)
    for n in range(NBINS):
        xn = x_ref[n]
        z = jnp.zeros_like(g)
        for a, w in _PLANE_TAPS[n]:
            z = z + jnp.where(q == a, w, 0.0)
        sp = jnp.maximum(xn, 0.0) + jnp.log1p(jnp.exp(-jnp.abs(xn)))
        inner = inner + (sp - xn * z)
    partial = jnp.sum(validf * inner).reshape(1, 1)

    @pl.when(pl.program_id(0) == 0)
    def _():
        out_ref[...] = jnp.zeros_like(out_ref)

    out_ref[...] += partial


@jax.jit
def kernel(preds, gt):
    xt = preds.transpose(2, 0, 1)
    out = pl.pallas_call(
        _loss_kernel,
        grid=(T // TT,),
        in_specs=[
            pl.BlockSpec((NBINS, B, TT), lambda j: (0, 0, j)),
            pl.BlockSpec((B, TT), lambda j: (0, j)),
        ],
        out_specs=pl.BlockSpec((1, 1), lambda j: (0, 0)),
        out_shape=jax.ShapeDtypeStruct((1, 1), jnp.float32),
    )(xt, gt)
    return out[0, 0]
